# baseline (speedup 1.0000x reference)
import functools

import jax
import jax.numpy as jnp
from jax.experimental import pallas as pl
from jax.experimental.pallas import tpu as pltpu

_NUM_GRAPHS = 16


def _leaky(v):
    return jnp.where(v > 0, v, 0.01 * v)


def _mm_kernel(a_ref, b_ref, bias_ref, o_ref, acc_ref, *, k_steps):
    @pl.when(pl.program_id(2) == 0)
    def _():
        acc_ref[...] = jnp.zeros_like(acc_ref)

    acc_ref[...] += jnp.dot(a_ref[...], b_ref[...],
                            preferred_element_type=jnp.float32)

    @pl.when(pl.program_id(2) == k_steps - 1)
    def _():
        o_ref[...] = _leaky(acc_ref[...] + bias_ref[...])


def _matmul_bias_leaky(a, b, bias, bm, bn, bk):
    m, k = a.shape
    _, n = b.shape
    grid = (m // bm, n // bn, k // bk)
    return pl.pallas_call(
        functools.partial(_mm_kernel, k_steps=grid[2]),
        grid=grid,
        in_specs=[
            pl.BlockSpec((bm, bk), lambda i, j, kk: (i, kk)),
            pl.BlockSpec((bk, bn), lambda i, j, kk: (kk, j)),
            pl.BlockSpec((1, bn), lambda i, j, kk: (0, j)),
        ],
        out_specs=pl.BlockSpec((bm, bn), lambda i, j, kk: (i, j)),
        out_shape=jax.ShapeDtypeStruct((m, n), jnp.float32),
        scratch_shapes=[pltpu.VMEM((bm, bn), jnp.float32)],
    )(a, b, bias.reshape(1, -1))


def _pool_kernel(h_ref, batch_ref, o_ref, *, num_graphs):
    i = pl.program_id(1)

    @pl.when(i == 0)
    def _():
        o_ref[...] = jnp.full_like(o_ref, -jnp.inf)

    h = h_ref[...]
    b = batch_ref[...]
    cols = []
    for g in range(num_graphs):
        mask = b == g
        cols.append(jnp.max(jnp.where(mask, h, -jnp.inf), axis=0))
    o_ref[...] = jnp.maximum(o_ref[...], jnp.stack(cols))


def _segment_max_pool(h, batch2d, bm, bn):
    m, n = h.shape
    grid = (n // bn, m // bm)
    return pl.pallas_call(
        functools.partial(_pool_kernel, num_graphs=_NUM_GRAPHS),
        grid=grid,
        in_specs=[
            pl.BlockSpec((bm, bn), lambda j, i: (i, j)),
            pl.BlockSpec((bm, 1), lambda j, i: (i, 0)),
        ],
        out_specs=pl.BlockSpec((_NUM_GRAPHS, bn), lambda j, i: (0, j)),
        out_shape=jax.ShapeDtypeStruct((_NUM_GRAPHS, n), jnp.float32),
    )(h, batch2d)


def _head_kernel(g_ref, w1_ref, b1_ref, w2_ref, b2_ref, o_ref, acc_ref,
                 *, k_steps):
    kk = pl.program_id(0)

    @pl.when(kk == 0)
    def _():
        acc_ref[...] = jnp.zeros_like(acc_ref)

    acc_ref[...] += jnp.dot(g_ref[...], w1_ref[...],
                            preferred_element_type=jnp.float32)

    @pl.when(kk == k_steps - 1)
    def _():
        g1 = _leaky(acc_ref[...] + b1_ref[...])
        o_ref[...] = _leaky(
            jnp.dot(g1, w2_ref[...], preferred_element_type=jnp.float32)
            + b2_ref[...])


def _mlp_head(g, l1W, l1b, l2W, l2b, bk):
    m, k = g.shape
    _, n1 = l1W.shape
    _, n2 = l2W.shape
    grid = (k // bk,)
    return pl.pallas_call(
        functools.partial(_head_kernel, k_steps=grid[0]),
        grid=grid,
        in_specs=[
            pl.BlockSpec((m, bk), lambda kk: (0, kk)),
            pl.BlockSpec((bk, n1), lambda kk: (kk, 0)),
            pl.BlockSpec((1, n1), lambda kk: (0, 0)),
            pl.BlockSpec((n1, n2), lambda kk: (0, 0)),
            pl.BlockSpec((1, n2), lambda kk: (0, 0)),
        ],
        out_specs=pl.BlockSpec((m, n2), lambda kk: (0, 0)),
        out_shape=jax.ShapeDtypeStruct((m, n2), jnp.float32),
        scratch_shapes=[pltpu.VMEM((m, n1), jnp.float32)],
    )(g, l1W, l1b.reshape(1, -1), l2W, l2b.reshape(1, -1))


def _pad_cols(a, n):
    return jnp.pad(a, ((0, 0), (0, n - a.shape[1])))


def kernel(x, edge_index, batch, embed, W1, b1, W2, b2, l1W, l1b, l2W, l2b):
    num_nodes = x.shape[0]

    src = edge_index[0]
    dst = edge_index[1]
    loop = jnp.arange(num_nodes, dtype=src.dtype)
    src = jnp.concatenate([src, loop])
    dst = jnp.concatenate([dst, loop])
    deg = jnp.zeros((num_nodes,), jnp.float32).at[dst].add(1.0)
    dinv = 1.0 / jnp.sqrt(deg)
    norm = (dinv[src] * dinv[dst])[:, None]

    def aggregate(h):
        msgs = jnp.take(h, src, axis=0) * norm
        return jax.ops.segment_sum(msgs, dst, num_segments=num_nodes)

    W1p = jnp.pad(W1, ((0, 96), (0, 144)))
    b1p = jnp.pad(b1, (0, 144))
    W2p = jnp.pad(W2, ((0, 144), (0, 144)))
    b2p = jnp.pad(b2, (0, 144))
    l1Wp = jnp.pad(l1W, ((0, 144), (0, 72)))
    l1bp = jnp.pad(l1b, (0, 72))
    l2Wp = jnp.pad(l2W, ((0, 72), (0, 0)))

    h = jnp.take(embed, x, axis=0).reshape(num_nodes, -1)

    h = _pad_cols(aggregate(h), 4096)
    h = _matmul_bias_leaky(h, W1p, b1p, bm=400, bn=1536, bk=2048)
    h = _matmul_bias_leaky(aggregate(h), W2p, b2p, bm=400, bn=1536, bk=2048)

    g = _segment_max_pool(h, batch.reshape(-1, 1), bm=400, bn=1536)
    return _mlp_head(g, l1Wp, l1bp, l2Wp, l2b, bk=1536)


# speedup vs baseline: 1.0665x; 1.0665x over previous
import functools

import jax
import jax.numpy as jnp
from jax.experimental import pallas as pl
from jax.experimental.pallas import tpu as pltpu

_NUM_GRAPHS = 16


def _leaky(v):
    return jnp.where(v > 0, v, 0.01 * v)


def _mm_kernel(a_ref, b_ref, bias_ref, o_ref, acc_ref, *, k_steps):
    @pl.when(pl.program_id(2) == 0)
    def _():
        acc_ref[...] = jnp.zeros_like(acc_ref)

    acc_ref[...] += jnp.dot(a_ref[...], b_ref[...],
                            preferred_element_type=jnp.float32)

    @pl.when(pl.program_id(2) == k_steps - 1)
    def _():
        o_ref[...] = _leaky(acc_ref[...] + bias_ref[...])


def _matmul_bias_leaky(a, b, bias, bm, bn, bk):
    m, k = a.shape
    _, n = b.shape
    grid = (m // bm, n // bn, k // bk)
    return pl.pallas_call(
        functools.partial(_mm_kernel, k_steps=grid[2]),
        grid=grid,
        in_specs=[
            pl.BlockSpec((bm, bk), lambda i, j, kk: (i, kk)),
            pl.BlockSpec((bk, bn), lambda i, j, kk: (kk, j)),
            pl.BlockSpec((1, bn), lambda i, j, kk: (0, j)),
        ],
        out_specs=pl.BlockSpec((bm, bn), lambda i, j, kk: (i, j)),
        out_shape=jax.ShapeDtypeStruct((m, n), jnp.float32),
        scratch_shapes=[pltpu.VMEM((bm, bn), jnp.float32)],
    )(a, b, bias.reshape(1, -1))


def _pool_kernel(h_ref, batch_ref, o_ref, *, num_graphs):
    i = pl.program_id(1)

    @pl.when(i == 0)
    def _():
        o_ref[...] = jnp.full_like(o_ref, -jnp.inf)

    h = h_ref[...]
    b = batch_ref[...]
    cols = []
    for g in range(num_graphs):
        mask = b == g
        cols.append(jnp.max(jnp.where(mask, h, -jnp.inf), axis=0))
    o_ref[...] = jnp.maximum(o_ref[...], jnp.stack(cols))


def _segment_max_pool(h, batch2d, bm, bn):
    m, n = h.shape
    grid = (n // bn, m // bm)
    return pl.pallas_call(
        functools.partial(_pool_kernel, num_graphs=_NUM_GRAPHS),
        grid=grid,
        in_specs=[
            pl.BlockSpec((bm, bn), lambda j, i: (i, j)),
            pl.BlockSpec((bm, 1), lambda j, i: (i, 0)),
        ],
        out_specs=pl.BlockSpec((_NUM_GRAPHS, bn), lambda j, i: (0, j)),
        out_shape=jax.ShapeDtypeStruct((_NUM_GRAPHS, n), jnp.float32),
    )(h, batch2d)


def _head_kernel(g_ref, w1_ref, b1_ref, w2_ref, b2_ref, o_ref, acc_ref,
                 *, k_steps):
    kk = pl.program_id(0)

    @pl.when(kk == 0)
    def _():
        acc_ref[...] = jnp.zeros_like(acc_ref)

    acc_ref[...] += jnp.dot(g_ref[...], w1_ref[...],
                            preferred_element_type=jnp.float32)

    @pl.when(kk == k_steps - 1)
    def _():
        g1 = _leaky(acc_ref[...] + b1_ref[...])
        o_ref[...] = _leaky(
            jnp.dot(g1, w2_ref[...], preferred_element_type=jnp.float32)
            + b2_ref[...])


def _mlp_head(g, l1W, l1b, l2W, l2b, bk):
    m, k = g.shape
    _, n1 = l1W.shape
    _, n2 = l2W.shape
    grid = (k // bk,)
    return pl.pallas_call(
        functools.partial(_head_kernel, k_steps=grid[0]),
        grid=grid,
        in_specs=[
            pl.BlockSpec((m, bk), lambda kk: (0, kk)),
            pl.BlockSpec((bk, n1), lambda kk: (kk, 0)),
            pl.BlockSpec((1, n1), lambda kk: (0, 0)),
            pl.BlockSpec((n1, n2), lambda kk: (0, 0)),
            pl.BlockSpec((1, n2), lambda kk: (0, 0)),
        ],
        out_specs=pl.BlockSpec((m, n2), lambda kk: (0, 0)),
        out_shape=jax.ShapeDtypeStruct((m, n2), jnp.float32),
        scratch_shapes=[pltpu.VMEM((m, n1), jnp.float32)],
    )(g, l1W, l1b.reshape(1, -1), l2W, l2b.reshape(1, -1))


def _pad_cols(a, n):
    return jnp.pad(a, ((0, 0), (0, n - a.shape[1])))


def kernel(x, edge_index, batch, embed, W1, b1, W2, b2, l1W, l1b, l2W, l2b):
    num_nodes = x.shape[0]

    src = edge_index[0]
    dst = edge_index[1]
    loop = jnp.arange(num_nodes, dtype=src.dtype)
    src = jnp.concatenate([src, loop])
    dst = jnp.concatenate([dst, loop])
    deg = jnp.zeros((num_nodes,), jnp.float32).at[dst].add(1.0)
    dinv = 1.0 / jnp.sqrt(deg)
    norm = (dinv[src] * dinv[dst])[:, None]

    def aggregate(h):
        msgs = jnp.take(h, src, axis=0) * norm
        return jax.ops.segment_sum(msgs, dst, num_segments=num_nodes)

    W1p = jnp.pad(W1, ((0, 96), (0, 144)))
    b1p = jnp.pad(b1, (0, 144))
    W2p = jnp.pad(W2, ((0, 144), (0, 144)))
    b2p = jnp.pad(b2, (0, 144))
    l1Wp = jnp.pad(l1W, ((0, 144), (0, 72)))
    l1bp = jnp.pad(l1b, (0, 72))
    l2Wp = jnp.pad(l2W, ((0, 72), (0, 0)))

    h = jnp.take(embed, x, axis=0).reshape(num_nodes, -1)

    h = _pad_cols(aggregate(h), 4096)
    h = _matmul_bias_leaky(h, W1p, b1p, bm=1000, bn=1536, bk=2048)
    h = _matmul_bias_leaky(aggregate(h), W2p, b2p, bm=1000, bn=1536, bk=2048)

    g = _segment_max_pool(h, batch.reshape(-1, 1), bm=400, bn=1536)
    return _mlp_head(g, l1Wp, l1bp, l2Wp, l2b, bk=1536)
